# baseline (speedup 1.0000x reference)
import functools

import jax
import jax.numpy as jnp
import numpy as np
from jax import lax
from jax.experimental import pallas as pl
from jax.experimental.pallas import tpu as pltpu
from jax.experimental.pallas import tpu_sc as plsc

B, T, D, FDIM = 8, 256, 768, 256
N_NEG = 100
LOGIT_TEMP = 0.1
NPAD = 112
NW = 32
ROWS_PER_W = T * B // NW
GROUPS = ROWS_PER_W // 16

_IDXT = None


def _rotl32(x, r):
    return ((x << np.uint32(r)) | (x >> np.uint32(32 - r))).astype(np.uint32)


def _threefry2x32(k1, k2, x0, x1):
    x0 = x0.astype(np.uint32).copy()
    x1 = x1.astype(np.uint32).copy()
    ks0, ks1 = np.uint32(k1), np.uint32(k2)
    ks2 = np.uint32(ks0 ^ ks1 ^ np.uint32(0x1BD11BDA))
    rot_a, rot_b = (13, 15, 26, 6), (17, 29, 16, 24)
    x0 += ks0
    x1 += ks1
    sched = ((rot_a, ks1, ks2, 1), (rot_b, ks2, ks0, 2), (rot_a, ks0, ks1, 3),
             (rot_b, ks1, ks2, 4), (rot_a, ks2, ks0, 5))
    for rots, a0, a1, i in sched:
        for r in rots:
            x0 += x1
            x1 = _rotl32(x1, r)
            x1 ^= x0
        x0 += a0
        x1 += a1 + np.uint32(i)
    return x0, x1


def _np_random_bits(k1, k2, size):
    c1 = np.zeros(size, np.uint32)
    c2 = np.arange(size, dtype=np.uint32)
    o0, o1 = _threefry2x32(k1, k2, c1, c2)
    return o0 ^ o1


def _np_randint(seed, shape, minval, maxval):
    k1, k2 = np.uint32(seed >> 32), np.uint32(seed & 0xFFFFFFFF)
    c1 = np.zeros(2, np.uint32)
    c2 = np.arange(2, dtype=np.uint32)
    b1, b2 = _threefry2x32(k1, k2, c1, c2)
    size = int(np.prod(shape))
    hi = _np_random_bits(b1[0], b2[0], size).reshape(shape)
    lo = _np_random_bits(b1[1], b2[1], size).reshape(shape)
    span = np.uint32(maxval - minval)
    mult = np.uint32(np.uint32(65536) % span)
    mult = np.uint32((np.uint64(mult) * np.uint64(mult)) % np.uint64(span))
    out = ((hi % span) * mult + (lo % span)) % span
    return (np.int32(minval) + out.astype(np.int32)).astype(np.int32)


def _neg_index_table():
    global _IDXT
    if _IDXT is None:
        tszs = np.repeat(np.arange(T), N_NEG)
        neg = _np_randint(1, (B, N_NEG * T), 0, T - 1)
        neg = (neg + (neg >= tszs[None, :])).astype(np.int32)
        idx = neg.reshape(B, T, N_NEG)
        diag = np.broadcast_to(np.arange(T, dtype=np.int32)[None, :, None], (B, T, NPAD - N_NEG))
        cols = np.concatenate([idx, diag], axis=2)
        p = np.arange(B * T).reshape(NW, GROUPS, 16)
        c = cols[p // T, p % T]
        flat = (p % ROWS_PER_W)[..., None] * T + c
        _IDXT = np.ascontiguousarray(np.transpose(flat, (0, 1, 3, 2))).reshape(NW, -1).astype(np.int32)
    return _IDXT


def _stage1(cnn_ref, mask_ref, qnt_ref, wp_ref, bp_ref, wf_ref, bf_ref, cm_ref, pos_ref):
    m = mask_ref[0]
    y = jnp.dot(cnn_ref[0] * m, wp_ref[...], preferred_element_type=jnp.float32) + bp_ref[...]
    x = jnp.dot(qnt_ref[0] * m, wf_ref[...], preferred_element_type=jnp.float32) + bf_ref[...]
    nt = (((1,), (1,)), ((), ()))
    g = lax.dot_general(y, y, nt, preferred_element_type=jnp.float32)
    s = lax.dot_general(x, y, nt, preferred_element_type=jnp.float32)
    rows = lax.broadcasted_iota(jnp.int32, (T, T), 0)
    cols = lax.broadcasted_iota(jnp.int32, (T, T), 1)
    eye = rows == cols
    gd_r = jnp.sum(jnp.where(eye, g, 0.0), axis=1, keepdims=True)
    gd_c = jnp.sum(jnp.where(eye, g, 0.0), axis=0, keepdims=True)
    d2 = gd_r + gd_c - 2.0 * g
    nx2 = jnp.sum(x * x, axis=1, keepdims=True)
    c = s / (jnp.maximum(jnp.sqrt(nx2 * gd_c), 1e-8) * LOGIT_TEMP)
    pos_ref[0] = jnp.sum(jnp.where(eye, c, 0.0), axis=0, keepdims=True)
    cm_ref[0] = jnp.where(d2 == 0.0, -jnp.inf, c)


def _sc_lse(cm_hbm, pos_hbm, idx_hbm, z_hbm, c_v, idx_v, pos_v, z_v):
    wid = lax.axis_index("s") * 2 + lax.axis_index("c")
    pltpu.sync_copy(cm_hbm.at[pl.ds(wid * (ROWS_PER_W * T), ROWS_PER_W * T)], c_v)
    pltpu.sync_copy(idx_hbm.at[wid], idx_v)
    pltpu.sync_copy(pos_hbm.at[pl.ds(wid * ROWS_PER_W, ROWS_PER_W)], pos_v)
    for grp in range(GROUPS):
        def zbody(j, z):
            iv = idx_v[pl.ds((grp * NPAD + j) * 16, 16)]
            return z + jnp.exp(plsc.load_gather(c_v, [iv]))

        z = lax.fori_loop(0, NPAD, zbody,
                          jnp.exp(pos_v[pl.ds(grp * 16, 16)]), unroll=8)
        z_v[pl.ds(grp * 16, 16)] = z
    pltpu.sync_copy(z_v, z_hbm.at[pl.ds(wid * ROWS_PER_W, ROWS_PER_W)])


def _stage3(z_ref, pos_ref, out_ref):
    out_ref[0, 0] = jnp.sum(jnp.log(z_ref[...]) - pos_ref[...])


def kernel(cnn_feat, mask_indices, quantized, W_proj_y, b_proj_y, W_final, b_final):
    maskf = mask_indices.astype(jnp.float32).reshape(B, T, 1)
    cm, pos = pl.pallas_call(
        _stage1,
        grid=(B,),
        in_specs=[
            pl.BlockSpec((1, T, D), lambda b: (b, 0, 0)),
            pl.BlockSpec((1, T, 1), lambda b: (b, 0, 0)),
            pl.BlockSpec((1, T, D), lambda b: (b, 0, 0)),
            pl.BlockSpec((D, FDIM), lambda b: (0, 0)),
            pl.BlockSpec((1, FDIM), lambda b: (0, 0)),
            pl.BlockSpec((D, FDIM), lambda b: (0, 0)),
            pl.BlockSpec((1, FDIM), lambda b: (0, 0)),
        ],
        out_specs=[
            pl.BlockSpec((1, T, T), lambda b: (b, 0, 0)),
            pl.BlockSpec((1, 1, T), lambda b: (b, 0, 0)),
        ],
        out_shape=[
            jax.ShapeDtypeStruct((B, T, T), jnp.float32),
            jax.ShapeDtypeStruct((B, 1, T), jnp.float32),
        ],
    )(cnn_feat, maskf, quantized, W_proj_y, b_proj_y.reshape(1, FDIM),
      W_final, b_final.reshape(1, FDIM))

    idxt = jnp.asarray(_neg_index_table())
    sc = pl.kernel(
        _sc_lse,
        mesh=plsc.VectorSubcoreMesh(core_axis_name="c", subcore_axis_name="s"),
        compiler_params=pltpu.CompilerParams(
            needs_layout_passes=False,
            skip_device_barrier=True,
            disable_bounds_checks=True,
            disable_semaphore_checks=True,
        ),
        out_type=jax.ShapeDtypeStruct((B * T,), jnp.float32),
        scratch_types=[
            pltpu.VMEM((ROWS_PER_W * T,), jnp.float32),
            pltpu.VMEM((GROUPS * NPAD * 16,), jnp.int32),
            pltpu.VMEM((ROWS_PER_W,), jnp.float32),
            pltpu.VMEM((ROWS_PER_W,), jnp.float32),
        ],
    )
    z = sc(cm.reshape(B * T * T), pos.reshape(B * T), idxt)

    out = pl.pallas_call(
        _stage3,
        out_shape=jax.ShapeDtypeStruct((1, 1), jnp.float32),
        out_specs=pl.BlockSpec(memory_space=pltpu.SMEM),
    )(z.reshape(16, 128), pos.reshape(16, 128))
    return out[0, 0]


# speedup vs baseline: 1.0229x; 1.0229x over previous
import functools

import jax
import jax.numpy as jnp
import numpy as np
from jax import lax
from jax.experimental import pallas as pl
from jax.experimental.pallas import tpu as pltpu
from jax.experimental.pallas import tpu_sc as plsc

B, T, D, FDIM = 8, 256, 768, 256
N_NEG = 100
LOGIT_TEMP = 0.1
NPAD = 112
NW = 32
ROWS_PER_W = T * B // NW
GROUPS = ROWS_PER_W // 16

_IDXT = None


def _rotl32(x, r):
    return ((x << np.uint32(r)) | (x >> np.uint32(32 - r))).astype(np.uint32)


def _threefry2x32(k1, k2, x0, x1):
    x0 = x0.astype(np.uint32).copy()
    x1 = x1.astype(np.uint32).copy()
    ks0, ks1 = np.uint32(k1), np.uint32(k2)
    ks2 = np.uint32(ks0 ^ ks1 ^ np.uint32(0x1BD11BDA))
    rot_a, rot_b = (13, 15, 26, 6), (17, 29, 16, 24)
    x0 += ks0
    x1 += ks1
    sched = ((rot_a, ks1, ks2, 1), (rot_b, ks2, ks0, 2), (rot_a, ks0, ks1, 3),
             (rot_b, ks1, ks2, 4), (rot_a, ks2, ks0, 5))
    for rots, a0, a1, i in sched:
        for r in rots:
            x0 += x1
            x1 = _rotl32(x1, r)
            x1 ^= x0
        x0 += a0
        x1 += a1 + np.uint32(i)
    return x0, x1


def _np_random_bits(k1, k2, size):
    c1 = np.zeros(size, np.uint32)
    c2 = np.arange(size, dtype=np.uint32)
    o0, o1 = _threefry2x32(k1, k2, c1, c2)
    return o0 ^ o1


def _np_randint(seed, shape, minval, maxval):
    k1, k2 = np.uint32(seed >> 32), np.uint32(seed & 0xFFFFFFFF)
    c1 = np.zeros(2, np.uint32)
    c2 = np.arange(2, dtype=np.uint32)
    b1, b2 = _threefry2x32(k1, k2, c1, c2)
    size = int(np.prod(shape))
    hi = _np_random_bits(b1[0], b2[0], size).reshape(shape)
    lo = _np_random_bits(b1[1], b2[1], size).reshape(shape)
    span = np.uint32(maxval - minval)
    mult = np.uint32(np.uint32(65536) % span)
    mult = np.uint32((np.uint64(mult) * np.uint64(mult)) % np.uint64(span))
    out = ((hi % span) * mult + (lo % span)) % span
    return (np.int32(minval) + out.astype(np.int32)).astype(np.int32)


def _neg_index_table():
    global _IDXT
    if _IDXT is None:
        tszs = np.repeat(np.arange(T), N_NEG)
        neg = _np_randint(1, (B, N_NEG * T), 0, T - 1)
        neg = (neg + (neg >= tszs[None, :])).astype(np.int32)
        idx = neg.reshape(B, T, N_NEG)
        diag = np.broadcast_to(np.arange(T, dtype=np.int32)[None, :, None], (B, T, NPAD - N_NEG))
        cols = np.concatenate([idx, diag], axis=2)
        p = np.arange(B * T).reshape(NW, GROUPS, 16)
        c = cols[p // T, p % T]
        flat = (p % ROWS_PER_W)[..., None] * T + c
        _IDXT = np.ascontiguousarray(np.transpose(flat, (0, 1, 3, 2))).reshape(NW, -1).astype(np.int32)
    return _IDXT


def _stage1(cnn_ref, mask_ref, qnt_ref, wp_ref, bp_ref, wf_ref, bf_ref, cm_ref, pos_ref):
    m = mask_ref[0]
    bf = jnp.bfloat16
    y = jnp.dot((cnn_ref[0] * m).astype(bf), wp_ref[...].astype(bf),
                preferred_element_type=jnp.float32) + bp_ref[...]
    x = jnp.dot((qnt_ref[0] * m).astype(bf), wf_ref[...].astype(bf),
                preferred_element_type=jnp.float32) + bf_ref[...]
    nt = (((1,), (1,)), ((), ()))
    yb, xb = y.astype(bf), x.astype(bf)
    g = lax.dot_general(yb, yb, nt, preferred_element_type=jnp.float32)
    s = lax.dot_general(xb, yb, nt, preferred_element_type=jnp.float32)
    rows = lax.broadcasted_iota(jnp.int32, (T, T), 0)
    cols = lax.broadcasted_iota(jnp.int32, (T, T), 1)
    eye = rows == cols
    gd_r = jnp.sum(jnp.where(eye, g, 0.0), axis=1, keepdims=True)
    gd_c = jnp.sum(jnp.where(eye, g, 0.0), axis=0, keepdims=True)
    d2 = gd_r + gd_c - 2.0 * g
    nx2 = jnp.sum(x * x, axis=1, keepdims=True)
    c = s / (jnp.maximum(jnp.sqrt(nx2 * gd_c), 1e-8) * LOGIT_TEMP)
    pos_ref[0] = jnp.sum(jnp.where(eye, c, 0.0), axis=0, keepdims=True)
    cm_ref[0] = jnp.where(d2 == 0.0, -jnp.inf, c)


def _sc_lse(cm_hbm, pos_hbm, idx_hbm, z_hbm, c_v, idx_v, pos_v, z_v, sem):
    wid = lax.axis_index("s") * 2 + lax.axis_index("c")
    h1 = pltpu.async_copy(cm_hbm.at[pl.ds(wid * (ROWS_PER_W * T), ROWS_PER_W * T)], c_v, sem)
    h2 = pltpu.async_copy(idx_hbm.at[wid], idx_v, sem)
    h3 = pltpu.async_copy(pos_hbm.at[pl.ds(wid * ROWS_PER_W, ROWS_PER_W)], pos_v, sem)
    h1.wait()
    h2.wait()
    h3.wait()
    for grp in range(GROUPS):
        def zbody(j, z):
            iv = idx_v[pl.ds((grp * NPAD + j) * 16, 16)]
            return z + jnp.exp(plsc.load_gather(c_v, [iv]))

        z = lax.fori_loop(0, NPAD, zbody,
                          jnp.exp(pos_v[pl.ds(grp * 16, 16)]), unroll=16)
        z_v[pl.ds(grp * 16, 16)] = z
    pltpu.sync_copy(z_v, z_hbm.at[pl.ds(wid * ROWS_PER_W, ROWS_PER_W)])


def _stage3(z_ref, pos_ref, out_ref):
    out_ref[0, 0] = jnp.sum(jnp.log(z_ref[...]) - pos_ref[...])


def kernel(cnn_feat, mask_indices, quantized, W_proj_y, b_proj_y, W_final, b_final):
    maskf = mask_indices.astype(jnp.float32).reshape(B, T, 1)
    cm, pos = pl.pallas_call(
        _stage1,
        grid=(B,),
        in_specs=[
            pl.BlockSpec((1, T, D), lambda b: (b, 0, 0)),
            pl.BlockSpec((1, T, 1), lambda b: (b, 0, 0)),
            pl.BlockSpec((1, T, D), lambda b: (b, 0, 0)),
            pl.BlockSpec((D, FDIM), lambda b: (0, 0)),
            pl.BlockSpec((1, FDIM), lambda b: (0, 0)),
            pl.BlockSpec((D, FDIM), lambda b: (0, 0)),
            pl.BlockSpec((1, FDIM), lambda b: (0, 0)),
        ],
        out_specs=[
            pl.BlockSpec((1, T, T), lambda b: (b, 0, 0)),
            pl.BlockSpec((1, 1, T), lambda b: (b, 0, 0)),
        ],
        out_shape=[
            jax.ShapeDtypeStruct((B, T, T), jnp.float32),
            jax.ShapeDtypeStruct((B, 1, T), jnp.float32),
        ],
    )(cnn_feat, maskf, quantized, W_proj_y, b_proj_y.reshape(1, FDIM),
      W_final, b_final.reshape(1, FDIM))

    idxt = jnp.asarray(_neg_index_table())
    sc = pl.kernel(
        _sc_lse,
        mesh=plsc.VectorSubcoreMesh(core_axis_name="c", subcore_axis_name="s"),
        compiler_params=pltpu.CompilerParams(needs_layout_passes=False),
        out_type=jax.ShapeDtypeStruct((B * T,), jnp.float32),
        scratch_types=[
            pltpu.VMEM((ROWS_PER_W * T,), jnp.float32),
            pltpu.VMEM((GROUPS * NPAD * 16,), jnp.int32),
            pltpu.VMEM((ROWS_PER_W,), jnp.float32),
            pltpu.VMEM((ROWS_PER_W,), jnp.float32),
            pltpu.SemaphoreType.DMA,
        ],
    )
    z = sc(cm.reshape(B * T * T), pos.reshape(B * T), idxt)

    out = pl.pallas_call(
        _stage3,
        out_shape=jax.ShapeDtypeStruct((1, 1), jnp.float32),
        out_specs=pl.BlockSpec(memory_space=pltpu.SMEM),
    )(z.reshape(16, 128), pos.reshape(16, 128))
    return out[0, 0]


# speedup vs baseline: 1.1373x; 1.1118x over previous
import functools

import jax
import jax.numpy as jnp
import numpy as np
from jax import lax
from jax.experimental import pallas as pl
from jax.experimental.pallas import tpu as pltpu
from jax.experimental.pallas import tpu_sc as plsc

B, T, D, FDIM = 8, 256, 768, 256
N_NEG = 100
LOGIT_TEMP = 0.1
NPAD = 112
NW = 32
ROWS_PER_W = T * B // NW
GROUPS = ROWS_PER_W // 16

_IDXT = None


def _rotl32(x, r):
    return ((x << np.uint32(r)) | (x >> np.uint32(32 - r))).astype(np.uint32)


def _threefry2x32(k1, k2, x0, x1):
    x0 = x0.astype(np.uint32).copy()
    x1 = x1.astype(np.uint32).copy()
    ks0, ks1 = np.uint32(k1), np.uint32(k2)
    ks2 = np.uint32(ks0 ^ ks1 ^ np.uint32(0x1BD11BDA))
    rot_a, rot_b = (13, 15, 26, 6), (17, 29, 16, 24)
    x0 += ks0
    x1 += ks1
    sched = ((rot_a, ks1, ks2, 1), (rot_b, ks2, ks0, 2), (rot_a, ks0, ks1, 3),
             (rot_b, ks1, ks2, 4), (rot_a, ks2, ks0, 5))
    for rots, a0, a1, i in sched:
        for r in rots:
            x0 += x1
            x1 = _rotl32(x1, r)
            x1 ^= x0
        x0 += a0
        x1 += a1 + np.uint32(i)
    return x0, x1


def _np_random_bits(k1, k2, size):
    c1 = np.zeros(size, np.uint32)
    c2 = np.arange(size, dtype=np.uint32)
    o0, o1 = _threefry2x32(k1, k2, c1, c2)
    return o0 ^ o1


def _np_randint(seed, shape, minval, maxval):
    k1, k2 = np.uint32(seed >> 32), np.uint32(seed & 0xFFFFFFFF)
    c1 = np.zeros(2, np.uint32)
    c2 = np.arange(2, dtype=np.uint32)
    b1, b2 = _threefry2x32(k1, k2, c1, c2)
    size = int(np.prod(shape))
    hi = _np_random_bits(b1[0], b2[0], size).reshape(shape)
    lo = _np_random_bits(b1[1], b2[1], size).reshape(shape)
    span = np.uint32(maxval - minval)
    mult = np.uint32(np.uint32(65536) % span)
    mult = np.uint32((np.uint64(mult) * np.uint64(mult)) % np.uint64(span))
    out = ((hi % span) * mult + (lo % span)) % span
    return (np.int32(minval) + out.astype(np.int32)).astype(np.int32)


def _neg_index_table():
    global _IDXT
    if _IDXT is None:
        tszs = np.repeat(np.arange(T), N_NEG)
        neg = _np_randint(1, (B, N_NEG * T), 0, T - 1)
        neg = (neg + (neg >= tszs[None, :])).astype(np.int32)
        idx = neg.reshape(B, T, N_NEG)
        diag = np.broadcast_to(np.arange(T, dtype=np.int32)[None, :, None], (B, T, NPAD - N_NEG))
        cols = np.concatenate([idx, diag], axis=2)
        p = np.arange(B * T).reshape(NW, GROUPS, 16)
        c = cols[p // T, p % T]
        flat = (p % ROWS_PER_W)[..., None] * T + c
        _IDXT = np.ascontiguousarray(np.transpose(flat, (0, 1, 3, 2))).reshape(NW, -1).astype(np.int32)
    return _IDXT


def _stage1(cnn_ref, mask_ref, qnt_ref, wp_ref, bp_ref, wf_ref, bf_ref, cm_ref, pos_ref):
    m = mask_ref[0]
    bf = jnp.bfloat16
    y = jnp.dot((cnn_ref[0] * m).astype(bf), wp_ref[...].astype(bf),
                preferred_element_type=jnp.float32) + bp_ref[...]
    x = jnp.dot((qnt_ref[0] * m).astype(bf), wf_ref[...].astype(bf),
                preferred_element_type=jnp.float32) + bf_ref[...]
    nt = (((1,), (1,)), ((), ()))
    yb, xb = y.astype(bf), x.astype(bf)
    g = lax.dot_general(yb, yb, nt, preferred_element_type=jnp.float32)
    s = lax.dot_general(xb, yb, nt, preferred_element_type=jnp.float32)
    rows = lax.broadcasted_iota(jnp.int32, (T, T), 0)
    cols = lax.broadcasted_iota(jnp.int32, (T, T), 1)
    eye = rows == cols
    gd_r = jnp.sum(jnp.where(eye, g, 0.0), axis=1, keepdims=True)
    gd_c = jnp.sum(jnp.where(eye, g, 0.0), axis=0, keepdims=True)
    d2 = gd_r + gd_c - 2.0 * g
    nx2 = jnp.sum(x * x, axis=1, keepdims=True)
    c = s / (jnp.maximum(jnp.sqrt(nx2 * gd_c), 1e-8) * LOGIT_TEMP)
    posv = jnp.sum(jnp.where(eye, c, 0.0), axis=0, keepdims=True)
    pos_ref[pl.ds(2 * pl.program_id(0), 2), :] = posv.reshape(2, 128)
    cm_ref[...] = jnp.where(d2 == 0.0, -jnp.inf, c).reshape(2 * T, 128)


def _sc_lse(cm_hbm, pos_hbm, idx_hbm, z_hbm, c_v, idx_v, pos_v, z_v, sem):
    wid = lax.axis_index("s") * 2 + lax.axis_index("c")
    h1 = pltpu.async_copy(cm_hbm.at[pl.ds(wid * (ROWS_PER_W * T), ROWS_PER_W * T)], c_v, sem)
    h2 = pltpu.async_copy(idx_hbm.at[wid], idx_v, sem)
    h3 = pltpu.async_copy(pos_hbm.at[pl.ds(wid * ROWS_PER_W, ROWS_PER_W)], pos_v, sem)
    h1.wait()
    h2.wait()
    h3.wait()
    for grp in range(GROUPS):
        def zbody(j, z):
            iv = idx_v[pl.ds((grp * NPAD + j) * 16, 16)]
            return z + jnp.exp(plsc.load_gather(c_v, [iv]))

        z = lax.fori_loop(0, NPAD, zbody,
                          jnp.exp(pos_v[pl.ds(grp * 16, 16)]), unroll=16)
        z_v[pl.ds(grp * 16, 16)] = z
    pltpu.sync_copy(z_v, z_hbm.at[pl.ds(wid * ROWS_PER_W, ROWS_PER_W)])


def _stage3(z_ref, pos_ref, out_ref):
    out_ref[0, 0] = jnp.sum(jnp.log(z_ref[...]) - pos_ref[...])


def kernel(cnn_feat, mask_indices, quantized, W_proj_y, b_proj_y, W_final, b_final):
    maskf = mask_indices.astype(jnp.float32).reshape(B, T, 1)
    cm, pos = pl.pallas_call(
        _stage1,
        grid=(B,),
        in_specs=[
            pl.BlockSpec((1, T, D), lambda b: (b, 0, 0)),
            pl.BlockSpec((1, T, 1), lambda b: (b, 0, 0)),
            pl.BlockSpec((1, T, D), lambda b: (b, 0, 0)),
            pl.BlockSpec((D, FDIM), lambda b: (0, 0)),
            pl.BlockSpec((1, FDIM), lambda b: (0, 0)),
            pl.BlockSpec((D, FDIM), lambda b: (0, 0)),
            pl.BlockSpec((1, FDIM), lambda b: (0, 0)),
        ],
        out_specs=[
            pl.BlockSpec((2 * T, 128), lambda b: (b, 0)),
            pl.BlockSpec((2 * B, 128), lambda b: (0, 0)),
        ],
        out_shape=[
            jax.ShapeDtypeStruct((B * 2 * T, 128), jnp.float32),
            jax.ShapeDtypeStruct((B * 2, 128), jnp.float32),
        ],
    )(cnn_feat, maskf, quantized, W_proj_y, b_proj_y.reshape(1, FDIM),
      W_final, b_final.reshape(1, FDIM))

    idxt = jnp.asarray(_neg_index_table())
    sc = pl.kernel(
        _sc_lse,
        mesh=plsc.VectorSubcoreMesh(core_axis_name="c", subcore_axis_name="s"),
        compiler_params=pltpu.CompilerParams(needs_layout_passes=False),
        out_type=jax.ShapeDtypeStruct((B * T,), jnp.float32),
        scratch_types=[
            pltpu.VMEM((ROWS_PER_W * T,), jnp.float32),
            pltpu.VMEM((GROUPS * NPAD * 16,), jnp.int32),
            pltpu.VMEM((ROWS_PER_W,), jnp.float32),
            pltpu.VMEM((ROWS_PER_W,), jnp.float32),
            pltpu.SemaphoreType.DMA,
        ],
    )
    z = sc(cm.reshape(B * T * T), pos.reshape(B * T), idxt)

    out = pl.pallas_call(
        _stage3,
        out_shape=jax.ShapeDtypeStruct((1, 1), jnp.float32),
        out_specs=pl.BlockSpec(memory_space=pltpu.SMEM),
    )(z.reshape(16, 128), pos)
    return out[0, 0]


# speedup vs baseline: 1.1508x; 1.0119x over previous
import functools

import jax
import jax.numpy as jnp
import numpy as np
from jax import lax
from jax.experimental import pallas as pl
from jax.experimental.pallas import tpu as pltpu
from jax.experimental.pallas import tpu_sc as plsc

B, T, D, FDIM = 8, 256, 768, 256
N_NEG = 100
LOGIT_TEMP = 0.1
NPAD = 112
NW = 32
ROWS_PER_W = T * B // NW
GROUPS = ROWS_PER_W // 16

_IDXT = None


def _rotl32(x, r):
    return ((x << np.uint32(r)) | (x >> np.uint32(32 - r))).astype(np.uint32)


def _threefry2x32(k1, k2, x0, x1):
    x0 = x0.astype(np.uint32).copy()
    x1 = x1.astype(np.uint32).copy()
    ks0, ks1 = np.uint32(k1), np.uint32(k2)
    ks2 = np.uint32(ks0 ^ ks1 ^ np.uint32(0x1BD11BDA))
    rot_a, rot_b = (13, 15, 26, 6), (17, 29, 16, 24)
    x0 += ks0
    x1 += ks1
    sched = ((rot_a, ks1, ks2, 1), (rot_b, ks2, ks0, 2), (rot_a, ks0, ks1, 3),
             (rot_b, ks1, ks2, 4), (rot_a, ks2, ks0, 5))
    for rots, a0, a1, i in sched:
        for r in rots:
            x0 += x1
            x1 = _rotl32(x1, r)
            x1 ^= x0
        x0 += a0
        x1 += a1 + np.uint32(i)
    return x0, x1


def _np_random_bits(k1, k2, size):
    c1 = np.zeros(size, np.uint32)
    c2 = np.arange(size, dtype=np.uint32)
    o0, o1 = _threefry2x32(k1, k2, c1, c2)
    return o0 ^ o1


def _np_randint(seed, shape, minval, maxval):
    k1, k2 = np.uint32(seed >> 32), np.uint32(seed & 0xFFFFFFFF)
    c1 = np.zeros(2, np.uint32)
    c2 = np.arange(2, dtype=np.uint32)
    b1, b2 = _threefry2x32(k1, k2, c1, c2)
    size = int(np.prod(shape))
    hi = _np_random_bits(b1[0], b2[0], size).reshape(shape)
    lo = _np_random_bits(b1[1], b2[1], size).reshape(shape)
    span = np.uint32(maxval - minval)
    mult = np.uint32(np.uint32(65536) % span)
    mult = np.uint32((np.uint64(mult) * np.uint64(mult)) % np.uint64(span))
    out = ((hi % span) * mult + (lo % span)) % span
    return (np.int32(minval) + out.astype(np.int32)).astype(np.int32)


def _neg_index_table():
    global _IDXT
    if _IDXT is None:
        tszs = np.repeat(np.arange(T), N_NEG)
        neg = _np_randint(1, (B, N_NEG * T), 0, T - 1)
        neg = (neg + (neg >= tszs[None, :])).astype(np.int32)
        idx = neg.reshape(B, T, N_NEG)
        diag = np.broadcast_to(np.arange(T, dtype=np.int32)[None, :, None], (B, T, NPAD - N_NEG))
        cols = np.concatenate([idx, diag], axis=2)
        p = np.arange(B * T).reshape(NW, GROUPS, 16)
        c = cols[p // T, p % T]
        flat = (p % ROWS_PER_W)[..., None] * T + c
        _IDXT = np.ascontiguousarray(np.transpose(flat, (0, 1, 3, 2))).reshape(NW, -1).astype(np.int32)
    return _IDXT


def _stage1(cnn_ref, mask_ref, qnt_ref, wp_ref, bp_ref, wf_ref, bf_ref, cm_ref, pos_ref):
    bf = jnp.bfloat16
    nt = (((1,), (1,)), ((), ()))
    wpb = wp_ref[...].astype(bf)
    wfb = wf_ref[...].astype(bf)
    rows = lax.broadcasted_iota(jnp.int32, (T, T), 0)
    cols = lax.broadcasted_iota(jnp.int32, (T, T), 1)
    eye = rows == cols
    for b in range(B):
        m = mask_ref[b]
        y = jnp.dot((cnn_ref[b] * m).astype(bf), wpb,
                    preferred_element_type=jnp.float32) + bp_ref[...]
        x = jnp.dot((qnt_ref[b] * m).astype(bf), wfb,
                    preferred_element_type=jnp.float32) + bf_ref[...]
        yb, xb = y.astype(bf), x.astype(bf)
        g = lax.dot_general(yb, yb, nt, preferred_element_type=jnp.float32)
        s = lax.dot_general(xb, yb, nt, preferred_element_type=jnp.float32)
        gd_r = jnp.sum(jnp.where(eye, g, 0.0), axis=1, keepdims=True)
        gd_c = jnp.sum(jnp.where(eye, g, 0.0), axis=0, keepdims=True)
        d2 = gd_r + gd_c - 2.0 * g
        nx2 = jnp.sum(x * x, axis=1, keepdims=True)
        c = s / (jnp.maximum(jnp.sqrt(nx2 * gd_c), 1e-8) * LOGIT_TEMP)
        posv = jnp.sum(jnp.where(eye, c, 0.0), axis=0, keepdims=True)
        pos_ref[pl.ds(2 * b, 2), :] = posv.reshape(2, 128)
        cm_ref[pl.ds(2 * T * b, 2 * T), :] = jnp.where(d2 == 0.0, -jnp.inf, c).reshape(2 * T, 128)


def _sc_lse(cm_hbm, pos_hbm, idx_hbm, z_hbm, c_v, idx_v, pos_v, z_v, sem):
    wid = lax.axis_index("s") * 2 + lax.axis_index("c")
    h1 = pltpu.async_copy(cm_hbm.at[pl.ds(wid * (ROWS_PER_W * T), ROWS_PER_W * T)], c_v, sem)
    h2 = pltpu.async_copy(idx_hbm.at[wid], idx_v, sem)
    h3 = pltpu.async_copy(pos_hbm.at[pl.ds(wid * ROWS_PER_W, ROWS_PER_W)], pos_v, sem)
    h1.wait()
    h2.wait()
    h3.wait()
    for grp in range(GROUPS):
        def zbody(j, z):
            iv = idx_v[pl.ds((grp * NPAD + j) * 16, 16)]
            return z + jnp.exp(plsc.load_gather(c_v, [iv]))

        z = lax.fori_loop(0, NPAD, zbody,
                          jnp.exp(pos_v[pl.ds(grp * 16, 16)]), unroll=16)
        z_v[pl.ds(grp * 16, 16)] = z
    pltpu.sync_copy(z_v, z_hbm.at[pl.ds(wid * ROWS_PER_W, ROWS_PER_W)])


def _stage3(z_ref, pos_ref, out_ref):
    out_ref[0, 0] = jnp.sum(jnp.log(z_ref[...]) - pos_ref[...])


def kernel(cnn_feat, mask_indices, quantized, W_proj_y, b_proj_y, W_final, b_final):
    maskf = mask_indices.astype(jnp.float32).reshape(B, T, 1)
    cm, pos = pl.pallas_call(
        _stage1,
        out_shape=[
            jax.ShapeDtypeStruct((B * 2 * T, 128), jnp.float32),
            jax.ShapeDtypeStruct((B * 2, 128), jnp.float32),
        ],
    )(cnn_feat, maskf, quantized, W_proj_y, b_proj_y.reshape(1, FDIM),
      W_final, b_final.reshape(1, FDIM))

    idxt = jnp.asarray(_neg_index_table())
    sc = pl.kernel(
        _sc_lse,
        mesh=plsc.VectorSubcoreMesh(core_axis_name="c", subcore_axis_name="s"),
        compiler_params=pltpu.CompilerParams(needs_layout_passes=False),
        out_type=jax.ShapeDtypeStruct((B * T,), jnp.float32),
        scratch_types=[
            pltpu.VMEM((ROWS_PER_W * T,), jnp.float32),
            pltpu.VMEM((GROUPS * NPAD * 16,), jnp.int32),
            pltpu.VMEM((ROWS_PER_W,), jnp.float32),
            pltpu.VMEM((ROWS_PER_W,), jnp.float32),
            pltpu.SemaphoreType.DMA,
        ],
    )
    z = sc(cm.reshape(B * T * T), pos.reshape(B * T), idxt)

    out = pl.pallas_call(
        _stage3,
        out_shape=jax.ShapeDtypeStruct((1, 1), jnp.float32),
        out_specs=pl.BlockSpec(memory_space=pltpu.SMEM),
    )(z.reshape(16, 128), pos)
    return out[0, 0]
